# baseline (speedup 1.0000x reference)
import functools

import jax
import jax.numpy as jnp
from jax import lax
from jax.experimental import pallas as pl
from jax.experimental.pallas import tpu as pltpu
from jax.experimental.pallas import tpu_sc as plsc

_NC = 2
_NS = 16
_NW = _NC * _NS
_CHUNK = 128


def _tc_transpose(t3, v_blk=4096):
    f_cat, emb, vocab = t3.shape
    quad = 128 // emb
    blks_per_f = -(-vocab // v_blk)
    v_pad = blks_per_f * v_blk
    rows_per_blk = v_blk * emb // 128
    n_rows = f_cat * blks_per_f * rows_per_blk

    def body(in_ref, out_ref):
        out_ref[...] = pltpu.einshape("a(bc)->b(ca)", in_ref[0], c=quad)

    table4 = pl.pallas_call(
        body,
        grid=(f_cat, blks_per_f),
        in_specs=[pl.BlockSpec((1, emb, v_blk), lambda f, j: (f, 0, j))],
        out_specs=pl.BlockSpec((rows_per_blk, 128),
                               lambda f, j: (f * blks_per_f + j, 0)),
        out_shape=jax.ShapeDtypeStruct((n_rows, 128), jnp.float32),
    )(t3)
    return table4, v_pad


def _sc_gather(table2d, idx3d, n_rows, emb_dim):
    chunks = idx3d.shape[1]
    rows_per_worker = chunks * _CHUNK
    mesh = plsc.VectorSubcoreMesh(core_axis_name="c", subcore_axis_name="s")

    @functools.partial(
        pl.kernel,
        out_type=jax.ShapeDtypeStruct((n_rows, emb_dim), jnp.float32),
        mesh=mesh,
        scratch_types=[
            pltpu.VMEM((chunks, _CHUNK), jnp.int32),
            pltpu.VMEM((_CHUNK, emb_dim), jnp.float32),
            pltpu.SemaphoreType.DMA,
        ],
        compiler_params=pltpu.CompilerParams(use_tc_tiling_on_sc=False),
    )
    def gather_kernel(table_hbm, idx_hbm, out_hbm, idx_v, rows_v, sem):
        wid = lax.axis_index("s") * _NC + lax.axis_index("c")
        pltpu.sync_copy(idx_hbm.at[wid], idx_v)
        base = pl.multiple_of(wid * rows_per_worker, _CHUNK)

        def body(j, carry):
            pltpu.async_copy(table_hbm.at[idx_v.at[j]], rows_v, sem).wait()
            pltpu.sync_copy(rows_v, out_hbm.at[pl.ds(base + j * _CHUNK, _CHUNK)])
            return carry

        lax.fori_loop(0, chunks, body, 0)

    return gather_kernel(table2d, idx3d)


def _tc_project(g2d, x_cont, w_full, bias_col, block_b):
    batch, k_g = g2d.shape
    k_c = x_cont.shape[1]
    n_out = w_full.shape[1]

    def body(g_ref, xc_ref, w_ref, b_ref, out_ref):
        g = g_ref[...].astype(jnp.bfloat16)
        x = xc_ref[...].astype(jnp.bfloat16)
        rhs = jnp.concatenate([g, x], axis=1)
        acc = jax.lax.dot_general(
            w_ref[...], rhs, (((0,), (1,)), ((), ())),
            preferred_element_type=jnp.float32)
        out_ref[...] = acc + b_ref[...]

    return pl.pallas_call(
        body,
        grid=(batch // block_b,),
        in_specs=[
            pl.BlockSpec((block_b, k_g), lambda i: (i, 0)),
            pl.BlockSpec((block_b, k_c), lambda i: (i, 0)),
            pl.BlockSpec((k_g + k_c, n_out), lambda i: (0, 0)),
            pl.BlockSpec((n_out, 1), lambda i: (0, 0)),
        ],
        out_specs=pl.BlockSpec((n_out, block_b), lambda i: (0, i)),
        out_shape=jax.ShapeDtypeStruct((n_out, batch), jnp.float32),
    )(g2d, x_cont, w_full, bias_col)


def kernel(x_cat, x_cont, cat_tables, cat_W, cat_b, cont_W, cont_b, cls_token):
    batch, f_cat = x_cat.shape
    f_cont = x_cont.shape[1]
    _, vocab, emb = cat_tables.shape
    d = cat_W.shape[2]
    f_pad = f_cat + 2

    t3 = jnp.transpose(cat_tables, (0, 2, 1))
    table4, v_pad = _tc_transpose(t3)
    table2d = table4.reshape(f_cat * v_pad, emb)

    idx = x_cat.astype(jnp.int32) + (jnp.arange(f_cat, dtype=jnp.int32) * v_pad)[None, :]
    idx = jnp.concatenate(
        [idx, jnp.zeros((batch, f_pad - f_cat), jnp.int32)], axis=1)
    n_rows = batch * f_pad
    idx3d = idx.reshape(_NW, n_rows // (_NW * _CHUNK), _CHUNK)

    gathered = _sc_gather(table2d, idx3d, n_rows, emb)
    g2d = gathered.reshape(batch, f_pad * emb)

    wdt = cat_W.dtype
    eye_cat = jnp.eye(f_cat, dtype=wdt)
    w_cat = (eye_cat[:, None, :, None] * cat_W[:, :, None, :]).reshape(f_cat * emb, f_cat * d)
    eye_cont = jnp.eye(f_cont, dtype=wdt)
    w_cont = (eye_cont[:, :, None] * cont_W[:, None, :]).reshape(f_cont, f_cont * d)
    n_out = (1 + f_cat + f_cont) * d
    top = jnp.concatenate(
        [jnp.zeros((f_cat * emb, d), wdt), w_cat, jnp.zeros((f_cat * emb, f_cont * d), wdt)],
        axis=1)
    mid = jnp.zeros(((f_pad - f_cat) * emb, n_out), wdt)
    bot = jnp.concatenate([jnp.zeros((f_cont, (1 + f_cat) * d), wdt), w_cont], axis=1)
    w_full = jnp.concatenate([top, mid, bot], axis=0).astype(jnp.bfloat16)
    bias_col = jnp.concatenate(
        [cls_token.reshape(d), cat_b.reshape(-1), cont_b.reshape(-1)]).reshape(n_out, 1)

    out_t = _tc_project(g2d, x_cont, w_full, bias_col, block_b=256)
    n_tok = 1 + f_cat + f_cont
    return out_t.reshape(n_tok, d, batch).transpose(2, 0, 1)


# speedup vs baseline: 5.8463x; 5.8463x over previous
import functools

import jax
import jax.numpy as jnp
from jax import lax
from jax.experimental import pallas as pl
from jax.experimental.pallas import tpu as pltpu
from jax.experimental.pallas import tpu_sc as plsc

_NC = 2
_NS = 16
_NW = _NC * _NS
_CHUNK = 128


def _tc_transpose(t3, v_blk=4096):
    f_cat, emb, vocab = t3.shape
    quad = 128 // emb
    blks_per_f = -(-vocab // v_blk)
    rows_per_blk = v_blk * emb // 128
    lanes_per_q = v_blk // quad
    n_rows = f_cat * blks_per_f * rows_per_blk
    eye = jnp.eye(128, dtype=jnp.float32)

    def body(in_ref, eye_ref, out_ref):
        x = in_ref[0]
        x4 = jnp.concatenate(
            [x[:, c * lanes_per_q:(c + 1) * lanes_per_q] for c in range(quad)],
            axis=0)
        out_ref[...] = jax.lax.dot_general(
            x4, eye_ref[...], (((0,), (0,)), ((), ())),
            preferred_element_type=jnp.float32)

    table4 = pl.pallas_call(
        body,
        grid=(f_cat, blks_per_f),
        in_specs=[pl.BlockSpec((1, emb, v_blk), lambda f, j: (f, 0, j)),
                  pl.BlockSpec((128, 128), lambda f, j: (0, 0))],
        out_specs=pl.BlockSpec((rows_per_blk, 128),
                               lambda f, j: (f * blks_per_f + j, 0)),
        out_shape=jax.ShapeDtypeStruct((n_rows, 128), jnp.float32),
    )(t3, eye)
    return table4, v_blk, blks_per_f


def _sc_gather(table2d, idx3d, n_rows, emb_dim):
    chunks = idx3d.shape[1]
    rows_per_worker = chunks * _CHUNK
    mesh = plsc.VectorSubcoreMesh(core_axis_name="c", subcore_axis_name="s")

    @functools.partial(
        pl.kernel,
        out_type=jax.ShapeDtypeStruct((n_rows, emb_dim), jnp.float32),
        mesh=mesh,
        scratch_types=[
            pltpu.VMEM((chunks, _CHUNK), jnp.int32),
            pltpu.VMEM((_CHUNK, emb_dim), jnp.float32),
            pltpu.SemaphoreType.DMA,
        ],
        compiler_params=pltpu.CompilerParams(use_tc_tiling_on_sc=False),
    )
    def gather_kernel(table_hbm, idx_hbm, out_hbm, idx_v, rows_v, sem):
        wid = lax.axis_index("s") * _NC + lax.axis_index("c")
        pltpu.sync_copy(idx_hbm.at[wid], idx_v)
        base = pl.multiple_of(wid * rows_per_worker, _CHUNK)

        def body(j, carry):
            pltpu.async_copy(table_hbm.at[idx_v.at[j]], rows_v, sem).wait()
            pltpu.sync_copy(rows_v, out_hbm.at[pl.ds(base + j * _CHUNK, _CHUNK)])
            return carry

        lax.fori_loop(0, chunks, body, 0)

    return gather_kernel(table2d, idx3d)


def _tc_project(g2d, x_cont, w_full, bias_col, block_b):
    batch, k_g = g2d.shape
    k_c = x_cont.shape[1]
    n_out = w_full.shape[1]

    def body(g_ref, xc_ref, w_ref, b_ref, out_ref):
        g = g_ref[...].astype(jnp.bfloat16)
        x = xc_ref[...].astype(jnp.bfloat16)
        rhs = jnp.concatenate([g, x], axis=1)
        acc = jax.lax.dot_general(
            w_ref[...], rhs, (((0,), (1,)), ((), ())),
            preferred_element_type=jnp.float32)
        out_ref[...] = acc + b_ref[...]

    return pl.pallas_call(
        body,
        grid=(batch // block_b,),
        in_specs=[
            pl.BlockSpec((block_b, k_g), lambda i: (i, 0)),
            pl.BlockSpec((block_b, k_c), lambda i: (i, 0)),
            pl.BlockSpec((k_g + k_c, n_out), lambda i: (0, 0)),
            pl.BlockSpec((n_out, 1), lambda i: (0, 0)),
        ],
        out_specs=pl.BlockSpec((n_out, block_b), lambda i: (0, i)),
        out_shape=jax.ShapeDtypeStruct((n_out, batch), jnp.float32),
    )(g2d, x_cont, w_full, bias_col)


def kernel(x_cat, x_cont, cat_tables, cat_W, cat_b, cont_W, cont_b, cls_token):
    batch, f_cat = x_cat.shape
    f_cont = x_cont.shape[1]
    _, vocab, emb = cat_tables.shape
    d = cat_W.shape[2]
    f_pad = f_cat + 2

    t3 = jnp.transpose(cat_tables, (0, 2, 1))
    table4, v_blk, blks_per_f = _tc_transpose(t3)
    table2d = table4.reshape(table4.shape[0] * (128 // emb), emb)

    lanes_per_q = v_blk // (128 // emb)
    v = x_cat.astype(jnp.int32)
    f_off = (jnp.arange(f_cat, dtype=jnp.int32) * blks_per_f)[None, :]
    ch = v // v_blk
    w = v % v_blk
    idx = (((f_off + ch) * lanes_per_q + w % lanes_per_q) * (128 // emb)
           + w // lanes_per_q)
    idx = jnp.concatenate(
        [idx, jnp.zeros((batch, f_pad - f_cat), jnp.int32)], axis=1)
    n_rows = batch * f_pad
    idx3d = idx.reshape(_NW, n_rows // (_NW * _CHUNK), _CHUNK)

    gathered = _sc_gather(table2d, idx3d, n_rows, emb)
    g2d = gathered.reshape(batch, f_pad * emb)

    wdt = cat_W.dtype
    eye_cat = jnp.eye(f_cat, dtype=wdt)
    w_cat = (eye_cat[:, None, :, None] * cat_W[:, :, None, :]).reshape(f_cat * emb, f_cat * d)
    eye_cont = jnp.eye(f_cont, dtype=wdt)
    w_cont = (eye_cont[:, :, None] * cont_W[:, None, :]).reshape(f_cont, f_cont * d)
    n_out = (1 + f_cat + f_cont) * d
    top = jnp.concatenate(
        [jnp.zeros((f_cat * emb, d), wdt), w_cat, jnp.zeros((f_cat * emb, f_cont * d), wdt)],
        axis=1)
    mid = jnp.zeros(((f_pad - f_cat) * emb, n_out), wdt)
    bot = jnp.concatenate([jnp.zeros((f_cont, (1 + f_cat) * d), wdt), w_cont], axis=1)
    w_full = jnp.concatenate([top, mid, bot], axis=0).astype(jnp.bfloat16)
    bias_col = jnp.concatenate(
        [cls_token.reshape(d), cat_b.reshape(-1), cont_b.reshape(-1)]).reshape(n_out, 1)

    out_t = _tc_project(g2d, x_cont, w_full, bias_col, block_b=256)
    n_tok = 1 + f_cat + f_cont
    return out_t.reshape(n_tok, d, batch).transpose(2, 0, 1)


# speedup vs baseline: 7.9962x; 1.3677x over previous
import functools

import jax
import jax.numpy as jnp
from jax import lax
from jax.experimental import pallas as pl
from jax.experimental.pallas import tpu as pltpu
from jax.experimental.pallas import tpu_sc as plsc

_NC = 2
_NS = 16
_NW = _NC * _NS
_CHUNK = 128


def _tc_transpose(t3, v_blk=4096):
    f_cat, emb, vocab = t3.shape
    quad = 128 // emb
    blks_per_f = -(-vocab // v_blk)
    rows_per_blk = v_blk * emb // 128
    lanes_per_q = v_blk // quad
    n_rows = f_cat * blks_per_f * rows_per_blk
    eye = jnp.eye(128, dtype=jnp.float32)

    def body(in_ref, eye_ref, out_ref):
        x = in_ref[0]
        x4 = jnp.concatenate(
            [x[:, c * lanes_per_q:(c + 1) * lanes_per_q] for c in range(quad)],
            axis=0)
        out_ref[...] = jax.lax.dot_general(
            x4, eye_ref[...], (((0,), (0,)), ((), ())),
            preferred_element_type=jnp.float32)

    table4 = pl.pallas_call(
        body,
        grid=(f_cat, blks_per_f),
        in_specs=[pl.BlockSpec((1, emb, v_blk), lambda f, j: (f, 0, j)),
                  pl.BlockSpec((128, 128), lambda f, j: (0, 0))],
        out_specs=pl.BlockSpec((rows_per_blk, 128),
                               lambda f, j: (f * blks_per_f + j, 0)),
        out_shape=jax.ShapeDtypeStruct((n_rows, 128), jnp.float32),
    )(t3, eye)
    return table4, v_blk, blks_per_f


def _sc_gather(table2d, idx3d, n_rows, emb_dim):
    chunks = idx3d.shape[1]
    rows_per_worker = chunks * _CHUNK
    mesh = plsc.VectorSubcoreMesh(core_axis_name="c", subcore_axis_name="s")

    @functools.partial(
        pl.kernel,
        out_type=jax.ShapeDtypeStruct((n_rows, emb_dim), jnp.float32),
        mesh=mesh,
        scratch_types=[
            pltpu.VMEM((chunks, _CHUNK), jnp.int32),
            pltpu.VMEM((_CHUNK, emb_dim), jnp.float32),
            pltpu.SemaphoreType.DMA,
        ],
        compiler_params=pltpu.CompilerParams(use_tc_tiling_on_sc=False),
    )
    def gather_kernel(table_hbm, idx_hbm, out_hbm, idx_v, rows_v, sem):
        wid = lax.axis_index("s") * _NC + lax.axis_index("c")
        pltpu.sync_copy(idx_hbm.at[wid], idx_v)
        base = pl.multiple_of(wid * rows_per_worker, _CHUNK)

        def body(j, carry):
            pltpu.async_copy(table_hbm.at[idx_v.at[j]], rows_v, sem).wait()
            pltpu.sync_copy(rows_v, out_hbm.at[pl.ds(base + j * _CHUNK, _CHUNK)])
            return carry

        lax.fori_loop(0, chunks, body, 0)

    return gather_kernel(table2d, idx3d)


def _tc_project(g2d, x_cont, w_full, bias_col, block_b):
    batch, k_g = g2d.shape
    k_c = x_cont.shape[1]
    n_out = w_full.shape[1]

    def body(g_ref, xc_ref, w_ref, b_ref, out_ref):
        g = g_ref[...].astype(jnp.bfloat16)
        x = xc_ref[...].astype(jnp.bfloat16)
        rhs = jnp.concatenate([g, x], axis=1)
        acc = jax.lax.dot_general(
            w_ref[...], rhs, (((0,), (1,)), ((), ())),
            preferred_element_type=jnp.float32)
        out_ref[...] = acc + b_ref[...]

    return pl.pallas_call(
        body,
        grid=(batch // block_b,),
        in_specs=[
            pl.BlockSpec((block_b, k_g), lambda i: (i, 0)),
            pl.BlockSpec((block_b, k_c), lambda i: (i, 0)),
            pl.BlockSpec((k_g + k_c, n_out), lambda i: (0, 0)),
            pl.BlockSpec((n_out, 1), lambda i: (0, 0)),
        ],
        out_specs=pl.BlockSpec((n_out, block_b), lambda i: (0, i)),
        out_shape=jax.ShapeDtypeStruct((n_out, batch), jnp.float32),
    )(g2d, x_cont, w_full, bias_col)


def kernel(x_cat, x_cont, cat_tables, cat_W, cat_b, cont_W, cont_b, cls_token):
    batch, f_cat = x_cat.shape
    f_cont = x_cont.shape[1]
    _, vocab, emb = cat_tables.shape
    d = cat_W.shape[2]
    f_pad = f_cat + 2

    t3 = jnp.transpose(cat_tables, (0, 2, 1))
    table4, v_blk, blks_per_f = _tc_transpose(t3)
    table2d = table4.reshape(table4.shape[0] * (128 // emb), emb)

    lanes_per_q = v_blk // (128 // emb)
    v = x_cat.astype(jnp.int32)
    f_off = (jnp.arange(f_cat, dtype=jnp.int32) * blks_per_f)[None, :]
    ch = v // v_blk
    w = v % v_blk
    idx = (((f_off + ch) * lanes_per_q + w % lanes_per_q) * (128 // emb)
           + w // lanes_per_q)
    idx = jnp.concatenate([idx, idx[:, :f_pad - f_cat]], axis=1)
    n_rows = batch * f_pad
    idx3d = idx.reshape(_NW, n_rows // (_NW * _CHUNK), _CHUNK)

    gathered = _sc_gather(table2d, idx3d, n_rows, emb)
    g2d = gathered.reshape(batch, f_pad * emb)

    wdt = cat_W.dtype
    eye_cat = jnp.eye(f_cat, dtype=wdt)
    w_cat = (eye_cat[:, None, :, None] * cat_W[:, :, None, :]).reshape(f_cat * emb, f_cat * d)
    eye_cont = jnp.eye(f_cont, dtype=wdt)
    w_cont = (eye_cont[:, :, None] * cont_W[:, None, :]).reshape(f_cont, f_cont * d)
    n_out = (1 + f_cat + f_cont) * d
    top = jnp.concatenate(
        [jnp.zeros((f_cat * emb, d), wdt), w_cat, jnp.zeros((f_cat * emb, f_cont * d), wdt)],
        axis=1)
    mid = jnp.zeros(((f_pad - f_cat) * emb, n_out), wdt)
    bot = jnp.concatenate([jnp.zeros((f_cont, (1 + f_cat) * d), wdt), w_cont], axis=1)
    w_full = jnp.concatenate([top, mid, bot], axis=0).astype(jnp.bfloat16)
    bias_col = jnp.concatenate(
        [cls_token.reshape(d), cat_b.reshape(-1), cont_b.reshape(-1)]).reshape(n_out, 1)

    out_t = _tc_project(g2d, x_cont, w_full, bias_col, block_b=256)
    n_tok = 1 + f_cat + f_cont
    return out_t.reshape(n_tok, d, batch).transpose(2, 0, 1)


# speedup vs baseline: 10.0246x; 1.2537x over previous
import functools

import jax
import jax.numpy as jnp
from jax import lax
from jax.experimental import pallas as pl
from jax.experimental.pallas import tpu as pltpu
from jax.experimental.pallas import tpu_sc as plsc

_NC = 2
_NS = 16
_NW = _NC * _NS
_CHUNK = 128


def _tc_transpose(t3, v_blk=8192):
    f_cat, emb, vocab = t3.shape
    quad = 128 // emb
    blks_per_f = -(-vocab // v_blk)
    rows_per_blk = v_blk * emb // 128
    lanes_per_q = v_blk // quad
    n_rows = f_cat * blks_per_f * rows_per_blk
    eye = jnp.eye(128, dtype=jnp.float32)

    def body(in_ref, eye_ref, out_ref):
        x = in_ref[0]
        x4 = jnp.concatenate(
            [x[:, c * lanes_per_q:(c + 1) * lanes_per_q] for c in range(quad)],
            axis=0)
        out_ref[...] = jax.lax.dot_general(
            x4, eye_ref[...], (((0,), (0,)), ((), ())),
            preferred_element_type=jnp.float32)

    table4 = pl.pallas_call(
        body,
        grid=(f_cat, blks_per_f),
        in_specs=[pl.BlockSpec((1, emb, v_blk), lambda f, j: (f, 0, j)),
                  pl.BlockSpec((128, 128), lambda f, j: (0, 0))],
        out_specs=pl.BlockSpec((rows_per_blk, 128),
                               lambda f, j: (f * blks_per_f + j, 0)),
        out_shape=jax.ShapeDtypeStruct((n_rows, 128), jnp.float32),
    )(t3, eye)
    return table4, v_blk, blks_per_f


def _sc_gather(table2d, idx3d, n_rows, emb_dim):
    chunks = idx3d.shape[1]
    rows_per_worker = chunks * _CHUNK
    mesh = plsc.VectorSubcoreMesh(core_axis_name="c", subcore_axis_name="s")

    @functools.partial(
        pl.kernel,
        out_type=jax.ShapeDtypeStruct((n_rows, emb_dim), jnp.float32),
        mesh=mesh,
        scratch_types=[
            pltpu.VMEM((chunks, _CHUNK), jnp.int32),
            pltpu.VMEM((_CHUNK, emb_dim), jnp.float32),
            pltpu.SemaphoreType.DMA,
        ],
        compiler_params=pltpu.CompilerParams(use_tc_tiling_on_sc=False),
    )
    def gather_kernel(table_hbm, idx_hbm, out_hbm, idx_v, rows_v, sem):
        wid = lax.axis_index("s") * _NC + lax.axis_index("c")
        pltpu.sync_copy(idx_hbm.at[wid], idx_v)
        base = pl.multiple_of(wid * rows_per_worker, _CHUNK)

        def body(j, carry):
            pltpu.async_copy(table_hbm.at[idx_v.at[j]], rows_v, sem).wait()
            pltpu.sync_copy(rows_v, out_hbm.at[pl.ds(base + j * _CHUNK, _CHUNK)])
            return carry

        lax.fori_loop(0, chunks, body, 0)

    return gather_kernel(table2d, idx3d)


def _tc_project(g2d, x_cont, w_full, bias_col, block_b):
    batch, k_g = g2d.shape
    k_c = x_cont.shape[1]
    n_out = w_full.shape[1]

    def body(g_ref, xc_ref, w_ref, b_ref, out_ref):
        g = g_ref[...].astype(jnp.bfloat16)
        x = xc_ref[...].astype(jnp.bfloat16)
        rhs = jnp.concatenate([g, x], axis=1)
        acc = jax.lax.dot_general(
            w_ref[...], rhs, (((0,), (1,)), ((), ())),
            preferred_element_type=jnp.float32)
        out_ref[...] = acc + b_ref[...]

    return pl.pallas_call(
        body,
        grid=(batch // block_b,),
        in_specs=[
            pl.BlockSpec((block_b, k_g), lambda i: (i, 0)),
            pl.BlockSpec((block_b, k_c), lambda i: (i, 0)),
            pl.BlockSpec((k_g + k_c, n_out), lambda i: (0, 0)),
            pl.BlockSpec((n_out, 1), lambda i: (0, 0)),
        ],
        out_specs=pl.BlockSpec((n_out, block_b), lambda i: (0, i)),
        out_shape=jax.ShapeDtypeStruct((n_out, batch), jnp.float32),
    )(g2d, x_cont, w_full, bias_col)


def kernel(x_cat, x_cont, cat_tables, cat_W, cat_b, cont_W, cont_b, cls_token):
    batch, f_cat = x_cat.shape
    f_cont = x_cont.shape[1]
    _, vocab, emb = cat_tables.shape
    d = cat_W.shape[2]
    f_pad = f_cat + 2

    t3 = jnp.transpose(cat_tables, (0, 2, 1))
    table4, v_blk, blks_per_f = _tc_transpose(t3)
    table2d = table4.reshape(table4.shape[0] * (128 // emb), emb)

    lanes_per_q = v_blk // (128 // emb)
    v = x_cat.astype(jnp.int32)
    f_off = (jnp.arange(f_cat, dtype=jnp.int32) * blks_per_f)[None, :]
    ch = v // v_blk
    w = v % v_blk
    idx = (((f_off + ch) * lanes_per_q + w % lanes_per_q) * (128 // emb)
           + w // lanes_per_q)
    idx = jnp.concatenate([idx, idx[:, :f_pad - f_cat]], axis=1)
    n_rows = batch * f_pad
    idx3d = idx.reshape(_NW, n_rows // (_NW * _CHUNK), _CHUNK)

    gathered = _sc_gather(table2d, idx3d, n_rows, emb)
    g2d = gathered.reshape(batch, f_pad * emb)

    wdt = cat_W.dtype
    eye_cat = jnp.eye(f_cat, dtype=wdt)
    w_cat = (eye_cat[:, None, :, None] * cat_W[:, :, None, :]).reshape(f_cat * emb, f_cat * d)
    eye_cont = jnp.eye(f_cont, dtype=wdt)
    w_cont = (eye_cont[:, :, None] * cont_W[:, None, :]).reshape(f_cont, f_cont * d)
    n_out = (1 + f_cat + f_cont) * d
    top = jnp.concatenate(
        [jnp.zeros((f_cat * emb, d), wdt), w_cat, jnp.zeros((f_cat * emb, f_cont * d), wdt)],
        axis=1)
    mid = jnp.zeros(((f_pad - f_cat) * emb, n_out), wdt)
    bot = jnp.concatenate([jnp.zeros((f_cont, (1 + f_cat) * d), wdt), w_cont], axis=1)
    w_full = jnp.concatenate([top, mid, bot], axis=0).astype(jnp.bfloat16)
    bias_col = jnp.concatenate(
        [cls_token.reshape(d), cat_b.reshape(-1), cont_b.reshape(-1)]).reshape(n_out, 1)

    out_t = _tc_project(g2d, x_cont, w_full, bias_col, block_b=512)
    n_tok = 1 + f_cat + f_cont
    return out_t.reshape(n_tok, d, batch).transpose(2, 0, 1)


# speedup vs baseline: 11.4077x; 1.1380x over previous
import functools

import jax
import jax.numpy as jnp
from jax import lax
from jax.experimental import pallas as pl
from jax.experimental.pallas import tpu as pltpu
from jax.experimental.pallas import tpu_sc as plsc

_NC = 2
_NS = 16
_NW = _NC * _NS
_CHUNK = 128


def _tc_transpose(t3, v_blk=16384):
    f_cat, emb, vocab = t3.shape
    quad = 128 // emb
    blks_per_f = -(-vocab // v_blk)
    rows_per_blk = v_blk * emb // 128
    lanes_per_q = v_blk // quad
    n_rows = f_cat * blks_per_f * rows_per_blk
    eye = jnp.eye(128, dtype=jnp.float32)

    def body(in_ref, eye_ref, out_ref):
        x = in_ref[0]
        x4 = jnp.concatenate(
            [x[:, c * lanes_per_q:(c + 1) * lanes_per_q] for c in range(quad)],
            axis=0)
        out_ref[...] = jax.lax.dot_general(
            x4, eye_ref[...], (((0,), (0,)), ((), ())),
            preferred_element_type=jnp.float32)

    table4 = pl.pallas_call(
        body,
        grid=(f_cat, blks_per_f),
        in_specs=[pl.BlockSpec((1, emb, v_blk), lambda f, j: (f, 0, j)),
                  pl.BlockSpec((128, 128), lambda f, j: (0, 0))],
        out_specs=pl.BlockSpec((rows_per_blk, 128),
                               lambda f, j: (f * blks_per_f + j, 0)),
        out_shape=jax.ShapeDtypeStruct((n_rows, 128), jnp.float32),
    )(t3, eye)
    return table4, v_blk, blks_per_f


def _sc_gather(table2d, idx3d, n_rows, emb_dim):
    chunks = idx3d.shape[1]
    rows_per_worker = chunks * _CHUNK
    mesh = plsc.VectorSubcoreMesh(core_axis_name="c", subcore_axis_name="s")

    @functools.partial(
        pl.kernel,
        out_type=jax.ShapeDtypeStruct((n_rows, emb_dim), jnp.float32),
        mesh=mesh,
        scratch_types=[
            pltpu.VMEM((chunks, _CHUNK), jnp.int32),
            pltpu.VMEM((_CHUNK, emb_dim), jnp.float32),
            pltpu.SemaphoreType.DMA,
        ],
        compiler_params=pltpu.CompilerParams(use_tc_tiling_on_sc=False),
    )
    def gather_kernel(table_hbm, idx_hbm, out_hbm, idx_v, rows_v, sem):
        wid = lax.axis_index("s") * _NC + lax.axis_index("c")
        pltpu.sync_copy(idx_hbm.at[wid], idx_v)
        base = pl.multiple_of(wid * rows_per_worker, _CHUNK)

        def body(j, carry):
            pltpu.async_copy(table_hbm.at[idx_v.at[j]], rows_v, sem).wait()
            pltpu.sync_copy(rows_v, out_hbm.at[pl.ds(base + j * _CHUNK, _CHUNK)])
            return carry

        lax.fori_loop(0, chunks, body, 0)

    return gather_kernel(table2d, idx3d)


def _tc_project(g2d, x_cont, w_full, bias_col, block_b):
    batch, k_g = g2d.shape
    k_c = x_cont.shape[1]
    n_out = w_full.shape[1]

    def body(g_ref, xc_ref, w_ref, b_ref, out_ref):
        g = g_ref[...].astype(jnp.bfloat16)
        x = xc_ref[...].astype(jnp.bfloat16)
        rhs = jnp.concatenate([g, x], axis=1)
        acc = jax.lax.dot_general(
            w_ref[...], rhs, (((0,), (1,)), ((), ())),
            preferred_element_type=jnp.float32)
        out_ref[...] = acc + b_ref[...]

    return pl.pallas_call(
        body,
        grid=(batch // block_b,),
        in_specs=[
            pl.BlockSpec((block_b, k_g), lambda i: (i, 0)),
            pl.BlockSpec((block_b, k_c), lambda i: (i, 0)),
            pl.BlockSpec((k_g + k_c, n_out), lambda i: (0, 0)),
            pl.BlockSpec((n_out, 1), lambda i: (0, 0)),
        ],
        out_specs=pl.BlockSpec((n_out, block_b), lambda i: (0, i)),
        out_shape=jax.ShapeDtypeStruct((n_out, batch), jnp.float32),
    )(g2d, x_cont, w_full, bias_col)


def kernel(x_cat, x_cont, cat_tables, cat_W, cat_b, cont_W, cont_b, cls_token):
    batch, f_cat = x_cat.shape
    f_cont = x_cont.shape[1]
    _, vocab, emb = cat_tables.shape
    d = cat_W.shape[2]
    f_pad = f_cat + 2

    t3 = jnp.transpose(cat_tables, (0, 2, 1))
    table4, v_blk, blks_per_f = _tc_transpose(t3)
    table2d = table4.reshape(table4.shape[0] * (128 // emb), emb)

    lanes_per_q = v_blk // (128 // emb)
    v = x_cat.astype(jnp.int32)
    f_off = (jnp.arange(f_cat, dtype=jnp.int32) * blks_per_f)[None, :]
    ch = v // v_blk
    w = v % v_blk
    idx = (((f_off + ch) * lanes_per_q + w % lanes_per_q) * (128 // emb)
           + w // lanes_per_q)
    idx = jnp.concatenate([idx, idx[:, :f_pad - f_cat]], axis=1)
    n_rows = batch * f_pad
    idx3d = idx.reshape(_NW, n_rows // (_NW * _CHUNK), _CHUNK)

    gathered = _sc_gather(table2d, idx3d, n_rows, emb)
    g2d = gathered.reshape(batch, f_pad * emb)

    wdt = cat_W.dtype
    eye_cat = jnp.eye(f_cat, dtype=wdt)
    w_cat = (eye_cat[:, None, :, None] * cat_W[:, :, None, :]).reshape(f_cat * emb, f_cat * d)
    eye_cont = jnp.eye(f_cont, dtype=wdt)
    w_cont = (eye_cont[:, :, None] * cont_W[:, None, :]).reshape(f_cont, f_cont * d)
    n_out = (1 + f_cat + f_cont) * d
    top = jnp.concatenate(
        [jnp.zeros((f_cat * emb, d), wdt), w_cat, jnp.zeros((f_cat * emb, f_cont * d), wdt)],
        axis=1)
    mid = jnp.zeros(((f_pad - f_cat) * emb, n_out), wdt)
    bot = jnp.concatenate([jnp.zeros((f_cont, (1 + f_cat) * d), wdt), w_cont], axis=1)
    w_full = jnp.concatenate([top, mid, bot], axis=0).astype(jnp.bfloat16)
    bias_col = jnp.concatenate(
        [cls_token.reshape(d), cat_b.reshape(-1), cont_b.reshape(-1)]).reshape(n_out, 1)

    out_t = _tc_project(g2d, x_cont, w_full, bias_col, block_b=512)
    n_tok = 1 + f_cat + f_cont
    return out_t.reshape(n_tok, d, batch).transpose(2, 0, 1)


# speedup vs baseline: 12.5483x; 1.1000x over previous
import functools

import jax
import jax.numpy as jnp
from jax import lax
from jax.experimental import pallas as pl
from jax.experimental.pallas import tpu as pltpu
from jax.experimental.pallas import tpu_sc as plsc

_NC = 2
_NS = 16
_NW = _NC * _NS
_CHUNK = 128


def _tc_transpose(t3, v_blk=20480):
    f_cat, emb, vocab = t3.shape
    quad = 128 // emb
    blks_per_f = -(-vocab // v_blk)
    rows_per_blk = v_blk * emb // 128
    lanes_per_q = v_blk // quad
    n_rows = f_cat * blks_per_f * rows_per_blk
    eye = jnp.eye(128, dtype=jnp.float32)

    def body(in_ref, eye_ref, out_ref):
        x = in_ref[0]
        x4 = jnp.concatenate(
            [x[:, c * lanes_per_q:(c + 1) * lanes_per_q] for c in range(quad)],
            axis=0)
        out_ref[...] = jax.lax.dot_general(
            x4, eye_ref[...], (((0,), (0,)), ((), ())),
            preferred_element_type=jnp.float32)

    table4 = pl.pallas_call(
        body,
        grid=(f_cat, blks_per_f),
        in_specs=[pl.BlockSpec((1, emb, v_blk), lambda f, j: (f, 0, j)),
                  pl.BlockSpec((128, 128), lambda f, j: (0, 0))],
        out_specs=pl.BlockSpec((rows_per_blk, 128),
                               lambda f, j: (f * blks_per_f + j, 0)),
        out_shape=jax.ShapeDtypeStruct((n_rows, 128), jnp.float32),
    )(t3, eye)
    return table4, v_blk, blks_per_f


def _sc_gather(table2d, idx3d, n_rows, emb_dim):
    chunks = idx3d.shape[1]
    rows_per_worker = chunks * _CHUNK
    mesh = plsc.VectorSubcoreMesh(core_axis_name="c", subcore_axis_name="s")

    @functools.partial(
        pl.kernel,
        out_type=jax.ShapeDtypeStruct((n_rows, emb_dim), jnp.float32),
        mesh=mesh,
        scratch_types=[
            pltpu.VMEM((chunks, _CHUNK), jnp.int32),
            pltpu.VMEM((_CHUNK, emb_dim), jnp.float32),
            pltpu.SemaphoreType.DMA,
        ],
        compiler_params=pltpu.CompilerParams(use_tc_tiling_on_sc=False),
    )
    def gather_kernel(table_hbm, idx_hbm, out_hbm, idx_v, rows_v, sem):
        wid = lax.axis_index("s") * _NC + lax.axis_index("c")
        pltpu.sync_copy(idx_hbm.at[wid], idx_v)
        base = pl.multiple_of(wid * rows_per_worker, _CHUNK)

        def body(j, carry):
            pltpu.async_copy(table_hbm.at[idx_v.at[j]], rows_v, sem).wait()
            pltpu.sync_copy(rows_v, out_hbm.at[pl.ds(base + j * _CHUNK, _CHUNK)])
            return carry

        lax.fori_loop(0, chunks, body, 0)

    return gather_kernel(table2d, idx3d)


def _tc_project(g2d, x_cont, w_full, bias_col, block_b):
    batch, k_g = g2d.shape
    k_c = x_cont.shape[1]
    n_out = w_full.shape[1]

    def body(g_ref, xc_ref, w_ref, b_ref, out_ref):
        g = g_ref[...].astype(jnp.bfloat16)
        x = xc_ref[...].astype(jnp.bfloat16)
        rhs = jnp.concatenate([g, x], axis=1)
        acc = jax.lax.dot_general(
            w_ref[...], rhs, (((0,), (1,)), ((), ())),
            preferred_element_type=jnp.float32)
        out_ref[...] = acc + b_ref[...]

    return pl.pallas_call(
        body,
        grid=(batch // block_b,),
        in_specs=[
            pl.BlockSpec((block_b, k_g), lambda i: (i, 0)),
            pl.BlockSpec((block_b, k_c), lambda i: (i, 0)),
            pl.BlockSpec((k_g + k_c, n_out), lambda i: (0, 0)),
            pl.BlockSpec((n_out, 1), lambda i: (0, 0)),
        ],
        out_specs=pl.BlockSpec((n_out, block_b), lambda i: (0, i)),
        out_shape=jax.ShapeDtypeStruct((n_out, batch), jnp.float32),
    )(g2d, x_cont, w_full, bias_col)


def kernel(x_cat, x_cont, cat_tables, cat_W, cat_b, cont_W, cont_b, cls_token):
    batch, f_cat = x_cat.shape
    f_cont = x_cont.shape[1]
    _, vocab, emb = cat_tables.shape
    d = cat_W.shape[2]
    f_pad = f_cat + 2

    t3 = jnp.transpose(cat_tables, (0, 2, 1))
    table4, v_blk, blks_per_f = _tc_transpose(t3)
    table2d = table4.reshape(table4.shape[0] * (128 // emb), emb)

    lanes_per_q = v_blk // (128 // emb)
    v = x_cat.astype(jnp.int32)
    f_off = (jnp.arange(f_cat, dtype=jnp.int32) * blks_per_f)[None, :]
    ch = v // v_blk
    w = v % v_blk
    idx = (((f_off + ch) * lanes_per_q + w % lanes_per_q) * (128 // emb)
           + w // lanes_per_q)
    idx = jnp.concatenate([idx, idx[:, :f_pad - f_cat]], axis=1)
    n_rows = batch * f_pad
    idx3d = idx.reshape(_NW, n_rows // (_NW * _CHUNK), _CHUNK)

    gathered = _sc_gather(table2d, idx3d, n_rows, emb)
    g2d = gathered.reshape(batch, f_pad * emb)

    wdt = cat_W.dtype
    eye_cat = jnp.eye(f_cat, dtype=wdt)
    w_cat = (eye_cat[:, None, :, None] * cat_W[:, :, None, :]).reshape(f_cat * emb, f_cat * d)
    eye_cont = jnp.eye(f_cont, dtype=wdt)
    w_cont = (eye_cont[:, :, None] * cont_W[:, None, :]).reshape(f_cont, f_cont * d)
    n_out = (1 + f_cat + f_cont) * d
    top = jnp.concatenate(
        [jnp.zeros((f_cat * emb, d), wdt), w_cat, jnp.zeros((f_cat * emb, f_cont * d), wdt)],
        axis=1)
    mid = jnp.zeros(((f_pad - f_cat) * emb, n_out), wdt)
    bot = jnp.concatenate([jnp.zeros((f_cont, (1 + f_cat) * d), wdt), w_cont], axis=1)
    w_full = jnp.concatenate([top, mid, bot], axis=0).astype(jnp.bfloat16)
    bias_col = jnp.concatenate(
        [cls_token.reshape(d), cat_b.reshape(-1), cont_b.reshape(-1)]).reshape(n_out, 1)

    out_t = _tc_project(g2d, x_cont, w_full, bias_col, block_b=512)
    n_tok = 1 + f_cat + f_cont
    return out_t.reshape(n_tok, d, batch).transpose(2, 0, 1)
